# baseline (speedup 1.0000x reference)
import jax
import jax.numpy as jnp
from jax import lax
from jax.experimental import pallas as pl
from jax.experimental.pallas import tpu as pltpu

_B, _T, _S, _H = 16, 4, 1025, 1280
_R = _B * _T * _S
_CH = 400
_NCH = _R // _CH
_K = 6


def _add_body(ids_ref, table_ref, hs_ref, out_ref, in_buf, out_buf, in_sem, out_sem):
    c = pl.program_id(0)
    slot = lax.rem(c, _K)

    def in_copy(chunk, sl):
        return pltpu.make_async_copy(
            hs_ref.at[pl.ds(chunk * _CH, _CH), :], in_buf.at[sl], in_sem.at[sl])

    def out_copy(chunk, sl):
        return pltpu.make_async_copy(
            out_buf.at[sl], out_ref.at[pl.ds(chunk * _CH, _CH), :], out_sem.at[sl])

    @pl.when(c == 0)
    def _prologue():
        for k in range(_K):
            in_copy(k, k).start()

    in_copy(c, slot).wait()

    @pl.when(c >= _K)
    def _drain_prev_out():
        out_copy(c - _K, slot).wait()

    r0 = c * _CH
    bt0 = r0 // _S
    bt1 = jnp.minimum(bt0 + 1, _B * _T - 1)
    b0, t0 = bt0 // _T, bt0 % _T
    b1, t1 = bt1 // _T, bt1 % _T
    j0 = ids_ref[b0] * _T + t0
    j1 = ids_ref[b1] * _T + t1
    bnd = (bt0 + 1) * _S - r0
    emb_a = table_ref[pl.ds(j0, 1), :]
    emb_b = table_ref[pl.ds(j1, 1), :]
    row = lax.broadcasted_iota(jnp.int32, (_CH, 1), 0)
    add = jnp.where(row < bnd, emb_a, emb_b)

    out_buf[slot] = in_buf[slot] + add

    out_copy(c, slot).start()

    @pl.when(c + _K < _NCH)
    def _prefetch_next():
        in_copy(c + _K, slot).start()

    @pl.when(c == _NCH - 1)
    def _drain_all_out():
        for k in range(_K):
            out_copy(0, k).wait()


def kernel(hidden_states, aspect_ratio_ids, embedding_weight):
    ids = aspect_ratio_ids.astype(jnp.int32)
    table = embedding_weight.reshape(-1, _H)
    hs2 = hidden_states.reshape(_R, _H)

    grid_spec = pltpu.PrefetchScalarGridSpec(
        num_scalar_prefetch=1,
        grid=(_NCH,),
        in_specs=[
            pl.BlockSpec((table.shape[0], _H), lambda c, ids_ref: (0, 0)),
            pl.BlockSpec(memory_space=pl.ANY),
        ],
        out_specs=pl.BlockSpec(memory_space=pl.ANY),
        scratch_shapes=[
            pltpu.VMEM((_K, _CH, _H), jnp.float32),
            pltpu.VMEM((_K, _CH, _H), jnp.float32),
            pltpu.SemaphoreType.DMA((_K,)),
            pltpu.SemaphoreType.DMA((_K,)),
        ],
    )
    out = pl.pallas_call(
        _add_body,
        grid_spec=grid_spec,
        out_shape=jax.ShapeDtypeStruct((_R, _H), jnp.float32),
    )(ids, table, hs2)
    return out.reshape(_B, _T, _S, _H)


# speedup vs baseline: 3.9259x; 3.9259x over previous
import jax
import jax.numpy as jnp
from jax import lax
from jax.experimental import pallas as pl
from jax.experimental.pallas import tpu as pltpu

_B, _T, _S, _H = 16, 4, 1025, 1280
_N = _B * _T
_K = 4


def _add_body(ids_ref, table_ref, hs_ref, out_ref, in_buf, out_buf, in_sem, out_sem):
    c = pl.program_id(0)
    slot = lax.rem(c, _K)

    def in_copy(chunk, sl):
        return pltpu.make_async_copy(
            hs_ref.at[chunk // _T, lax.rem(chunk, _T)], in_buf.at[sl], in_sem.at[sl])

    def out_copy(chunk, sl):
        return pltpu.make_async_copy(
            out_buf.at[sl], out_ref.at[chunk // _T, lax.rem(chunk, _T)], out_sem.at[sl])

    @pl.when(c == 0)
    def _prologue():
        for k in range(_K):
            in_copy(k, k).start()

    in_copy(c, slot).wait()

    @pl.when(c >= _K)
    def _drain_prev_out():
        out_copy(c - _K, slot).wait()

    j = ids_ref[c // _T] * _T + lax.rem(c, _T)
    out_buf[slot] = in_buf[slot] + table_ref[pl.ds(j, 1), :]

    out_copy(c, slot).start()

    @pl.when(c + _K < _N)
    def _prefetch_next():
        in_copy(c + _K, slot).start()

    @pl.when(c == _N - 1)
    def _drain_all_out():
        for k in range(_K):
            out_copy(0, k).wait()


def kernel(hidden_states, aspect_ratio_ids, embedding_weight):
    ids = aspect_ratio_ids.astype(jnp.int32)
    table = embedding_weight.reshape(-1, _H)

    grid_spec = pltpu.PrefetchScalarGridSpec(
        num_scalar_prefetch=1,
        grid=(_N,),
        in_specs=[
            pl.BlockSpec((table.shape[0], _H), lambda c, ids_ref: (0, 0)),
            pl.BlockSpec(memory_space=pl.ANY),
        ],
        out_specs=pl.BlockSpec(memory_space=pl.ANY),
        scratch_shapes=[
            pltpu.VMEM((_K, _S, _H), jnp.float32),
            pltpu.VMEM((_K, _S, _H), jnp.float32),
            pltpu.SemaphoreType.DMA((_K,)),
            pltpu.SemaphoreType.DMA((_K,)),
        ],
    )
    return pl.pallas_call(
        _add_body,
        grid_spec=grid_spec,
        out_shape=jax.ShapeDtypeStruct((_B, _T, _S, _H), jnp.float32),
    )(ids, table, hidden_states)
